# baseline (speedup 1.0000x reference)
import functools

import jax
import jax.numpy as jnp
from jax import lax
from jax.experimental import pallas as pl
from jax.experimental.pallas import tpu as pltpu
from jax.experimental.pallas import tpu_sc as plsc

N_ROWS = 100000
RAW_COLS = 64
N_COL = 128
BATCH = 16384

_info = plsc.get_sparse_core_info()
NC = _info.num_cores
NS = _info.num_subcores
L = _info.num_lanes
NW = NC * NS
BPW = BATCH // NW
G = BPW // L

_mesh = plsc.VectorSubcoreMesh(core_axis_name="c", subcore_axis_name="s")

@functools.partial(
    pl.kernel,
    mesh=_mesh,
    compiler_params=pltpu.CompilerParams(needs_layout_passes=False),
    out_type=jax.ShapeDtypeStruct((BATCH, N_COL), jnp.float32),
    scratch_types=[
        pltpu.VMEM((BPW,), jnp.int32),
        pltpu.VMEM((BPW, N_COL), jnp.float32),
        pltpu.VMEM((BPW // 2, N_COL), jnp.float32),
        pltpu.SemaphoreType.DMA,
        pltpu.SemaphoreType.DMA,
        pltpu.SemaphoreType.DMA,
        pltpu.SemaphoreType.DMA,
        pltpu.SemaphoreType.DMA,
        pltpu.SemaphoreType.DMA,
        pltpu.SemaphoreType.DMA,
        pltpu.SemaphoreType.DMA,
        pltpu.SemaphoreType.DMA,
    ],
)
def _lookup(table_hbm, idx_hbm, out_hbm, idx_v, rows_v, drain_v, s0, s1, s2, s3, s4, s5, s6, s7, sem2):
    wid = lax.axis_index("s") * NC + lax.axis_index("c")
    base = wid * BPW
    iota = lax.iota(jnp.int32, L)

    pltpu.sync_copy(idx_hbm.at[pl.ds(base, BPW)], idx_v)

    sems = [s0, s1, s2, s3, s4, s5, s6, s7]
    NCH = 8
    GPC = G // NCH
    RPC = BPW // NCH

    def make_issue(sem):
        def issue(g, carry):
            v = idx_v[pl.ds(g * L, L)]
            for k in range(L):
                pltpu.async_copy(
                    table_hbm.at[v[k]],
                    rows_v.at[g * L + k, pl.ds(0, RAW_COLS)],
                    sem,
                )
            return carry

        return issue

    for ch in range(NCH):
        lax.fori_loop(ch * GPC, (ch + 1) * GPC, make_issue(sems[ch]), 0)

    def fix(g, carry):
        rowidx = g * L + iota
        last = plsc.load_gather(
            rows_v, [rowidx, jnp.full((L,), RAW_COLS - 1, jnp.int32)]
        )
        for c in range(RAW_COLS):
            col = c + iota
            col = jnp.where(col >= RAW_COLS, col - RAW_COLS, col)
            plsc.store_scatter(rows_v, [rowidx, col + RAW_COLS], last)
        return carry

    for ch in range(NCH):
        pltpu.make_async_copy(
            out_hbm.at[pl.ds(0, RPC // 2)],
            drain_v.at[pl.ds(0, RPC // 2)],
            sems[ch],
        ).wait()
        lax.fori_loop(ch * GPC, (ch + 1) * GPC, fix, 0)
        pltpu.async_copy(
            rows_v.at[pl.ds(ch * RPC, RPC)],
            out_hbm.at[pl.ds(base + ch * RPC, RPC)],
            sem2,
        )

    pltpu.make_async_copy(
        out_hbm.at[pl.ds(0, BPW)], rows_v, sem2
    ).wait()


def kernel(table, index):
    return _lookup(table, index)


# speedup vs baseline: 1.0198x; 1.0198x over previous
import functools

import jax
import jax.numpy as jnp
from jax import lax
from jax.experimental import pallas as pl
from jax.experimental.pallas import tpu as pltpu
from jax.experimental.pallas import tpu_sc as plsc

N_ROWS = 100000
RAW_COLS = 64
N_COL = 128
BATCH = 16384

_info = plsc.get_sparse_core_info()
NC = _info.num_cores
NS = _info.num_subcores
L = _info.num_lanes
NW = NC * NS
BPW = BATCH // NW
G = BPW // L

_mesh = plsc.VectorSubcoreMesh(core_axis_name="c", subcore_axis_name="s")

@functools.partial(
    pl.kernel,
    mesh=_mesh,
    compiler_params=pltpu.CompilerParams(needs_layout_passes=False),
    out_type=jax.ShapeDtypeStruct((BATCH, N_COL), jnp.float32),
    scratch_types=[
        pltpu.VMEM((BPW,), jnp.int32),
        pltpu.VMEM((BPW, N_COL), jnp.float32),
        pltpu.VMEM((BPW // 8, N_COL), jnp.float32),
        pltpu.SemaphoreType.DMA,
        pltpu.SemaphoreType.DMA,
        pltpu.SemaphoreType.DMA,
        pltpu.SemaphoreType.DMA,
        pltpu.SemaphoreType.DMA,
    ],
)
def _lookup(table_hbm, idx_hbm, out_hbm, idx_v, rows_v, drain_v, s0, s1, s2, s3, sem2):
    wid = lax.axis_index("s") * NC + lax.axis_index("c")
    base = wid * BPW
    iota = lax.iota(jnp.int32, L)

    pltpu.sync_copy(idx_hbm.at[pl.ds(base, BPW)], idx_v)

    sems = [s0, s1, s2, s3]
    NCH = 4
    GPC = G // NCH
    RPC = BPW // NCH

    def make_issue(sem):
        def issue(g, carry):
            v = idx_v[pl.ds(g * L, L)]
            for k in range(L):
                pltpu.async_copy(
                    table_hbm.at[v[k]],
                    rows_v.at[g * L + k, pl.ds(0, RAW_COLS)],
                    sem,
                )
            return carry

        return issue

    for ch in range(NCH):
        lax.fori_loop(ch * GPC, (ch + 1) * GPC, make_issue(sems[ch]), 0)

    def fix(g, carry):
        rowidx = g * L + iota
        last = plsc.load_gather(
            rows_v, [rowidx, jnp.full((L,), RAW_COLS - 1, jnp.int32)]
        )
        for c in range(RAW_COLS):
            col = c + iota
            col = jnp.where(col >= RAW_COLS, col - RAW_COLS, col)
            plsc.store_scatter(rows_v, [rowidx, col + RAW_COLS], last)
        return carry

    for ch in range(NCH):
        pltpu.make_async_copy(
            out_hbm.at[pl.ds(0, RPC // 2)], drain_v, sems[ch]
        ).wait()
        lax.fori_loop(ch * GPC, (ch + 1) * GPC, fix, 0)
        pltpu.async_copy(
            rows_v.at[pl.ds(ch * RPC, RPC)],
            out_hbm.at[pl.ds(base + ch * RPC, RPC)],
            sem2,
        )

    pltpu.make_async_copy(
        out_hbm.at[pl.ds(0, BPW)], rows_v, sem2
    ).wait()


def kernel(table, index):
    return _lookup(table, index)
